# baseline (speedup 1.0000x reference)
import functools

import jax
import jax.numpy as jnp
from jax import lax
from jax.experimental import pallas as pl
from jax.experimental.pallas import tpu as pltpu
from jax.experimental.pallas import tpu_sc as plsc

N = 10000
E = 320000
D = 128

NC = 2
NS = 16
NW = NC * NS
CHUNK = 80
NBUF = 3
EPW = E // NW
NCHUNK = EPW // CHUNK
NACC = 10112
RPT = NACC // NS
LAST = N - 15 * RPT


def _sc_segment_sum(x, edges, zero_init):
    mesh = plsc.VectorSubcoreMesh(core_axis_name="c", subcore_axis_name="s")

    @functools.partial(
        pl.kernel,
        mesh=mesh,
        out_type=jax.ShapeDtypeStruct((2 * N, D), jnp.float32),
        scratch_types=[
            pltpu.VMEM((EPW,), jnp.int32),
            pltpu.VMEM_SHARED((NACC, D), jnp.float32),
        ]
        + [pltpu.VMEM((CHUNK, D), jnp.float32) for _ in range(NBUF)]
        + [pltpu.VMEM((CHUNK,), jnp.int32) for _ in range(NBUF)]
        + [pltpu.SemaphoreType.DMA for _ in range(3 * NBUF + 1)],
    )
    def k(x_hbm, e_hbm, zero_hbm, out_hbm, src_all, acc, *bufs):
        rows = bufs[:NBUF]
        dst_v = bufs[NBUF:2 * NBUF]
        gsem = bufs[2 * NBUF:3 * NBUF]
        dsem = bufs[3 * NBUF:4 * NBUF]
        ssem = bufs[4 * NBUF:5 * NBUF]
        zsem = bufs[5 * NBUF]
        c = lax.axis_index("c")
        s = lax.axis_index("s")
        w = s * NC + c

        zcopy = pltpu.async_copy(zero_hbm, acc.at[pl.ds(s * RPT, RPT)], zsem)
        ebase = w * EPW
        pltpu.sync_copy(e_hbm.at[pl.ds(ebase, EPW)], src_all)

        def start_chunk(j, u, reuse=True):
            if reuse:
                pltpu.make_async_copy(
                    rows[u], acc.at[dst_v[u]], ssem[u]).wait()
            base = pl.multiple_of(j * CHUNK, 8)
            pltpu.async_copy(e_hbm.at[pl.ds(E + ebase + base, CHUNK)],
                             dst_v[u], dsem[u])
            pltpu.async_copy(x_hbm.at[src_all.at[pl.ds(base, CHUNK)]],
                             rows[u], gsem[u])

        def finish_chunk(u):
            pltpu.make_async_copy(
                e_hbm.at[pl.ds(0, CHUNK)], dst_v[u], dsem[u]).wait()
            pltpu.make_async_copy(
                x_hbm.at[pl.ds(0, CHUNK)], rows[u], gsem[u]).wait()
            pltpu.async_copy(rows[u], acc.at[dst_v[u]], ssem[u], add=True)

        for u in range(NBUF):
            start_chunk(u, u, reuse=False)
        zcopy.wait()
        plsc.subcore_barrier()

        nfull = (NCHUNK - NBUF) // NBUF

        def body(i, carry):
            for u in range(NBUF):
                j = i * NBUF + u
                finish_chunk(u)
                start_chunk(j + NBUF, u)
            return carry

        lax.fori_loop(0, nfull, body, 0)
        for j in range(nfull * NBUF, NCHUNK):
            u = j % NBUF
            finish_chunk(u)
            if j + NBUF < NCHUNK:
                start_chunk(j + NBUF, u)
        for u in range(NBUF):
            pltpu.make_async_copy(rows[u], acc.at[dst_v[u]], ssem[u]).wait()
        plsc.subcore_barrier()

        @pl.when(s < NS - 1)
        def _():
            pltpu.sync_copy(acc.at[pl.ds(s * RPT, RPT)],
                            out_hbm.at[pl.ds(c * N + s * RPT, RPT)])

        @pl.when(s == NS - 1)
        def _():
            pltpu.sync_copy(acc.at[pl.ds((NS - 1) * RPT, LAST)],
                            out_hbm.at[pl.ds(c * N + (NS - 1) * RPT, LAST)])

    return k(x, edges, zero_init)


BM = 2000


def _tc_tail(x, partials, W, b2):
    def body(x_ref, p0_ref, p1_ref, w_ref, b_ref, o_ref):
        a = p0_ref[...] + p1_ref[...]
        h = jnp.dot(a, w_ref[...], preferred_element_type=jnp.float32)
        o_ref[...] = x_ref[...] + jnp.maximum(h + b_ref[...], 0.0)

    return pl.pallas_call(
        body,
        grid=(N // BM,),
        in_specs=[
            pl.BlockSpec((BM, D), lambda i: (i, 0)),
            pl.BlockSpec((BM, D), lambda i: (i, 0)),
            pl.BlockSpec((BM, D), lambda i: (i + N // BM, 0)),
            pl.BlockSpec((D, D), lambda i: (0, 0)),
            pl.BlockSpec((1, D), lambda i: (0, 0)),
        ],
        out_specs=pl.BlockSpec((BM, D), lambda i: (i, 0)),
        out_shape=jax.ShapeDtypeStruct((N, D), jnp.float32),
    )(x, partials, partials, W, b2)


def kernel(input, edge_index, cell_dropout, layer_dropout, node_lastlayer,
           stage1_flag, W, b):
    edges = edge_index.reshape(2 * E)
    zero_init = jnp.zeros((RPT, D), dtype=jnp.float32)

    partials = _sc_segment_sum(input, edges, zero_init)
    return _tc_tail(input, partials, W, b.reshape(1, D))


# speedup vs baseline: 1.0098x; 1.0098x over previous
import functools

import jax
import jax.numpy as jnp
from jax import lax
from jax.experimental import pallas as pl
from jax.experimental.pallas import tpu as pltpu
from jax.experimental.pallas import tpu_sc as plsc

N = 10000
E = 320000
D = 128

NC = 2
NS = 16
NW = NC * NS
CHUNK = 80
NBUF = 3
EPW = E // NW
NCHUNK = EPW // CHUNK
NACC = 10112
RPT = NACC // NS
LAST = N - 15 * RPT


def _sc_segment_sum(x, edges, zero_init):
    mesh = plsc.VectorSubcoreMesh(core_axis_name="c", subcore_axis_name="s")

    @functools.partial(
        pl.kernel,
        mesh=mesh,
        out_type=jax.ShapeDtypeStruct((2 * N, D), jnp.float32),
        scratch_types=[
            pltpu.VMEM((EPW,), jnp.int32),
            pltpu.VMEM_SHARED((NACC, D), jnp.float32),
        ]
        + [pltpu.VMEM((CHUNK, D), jnp.float32) for _ in range(NBUF)]
        + [pltpu.VMEM((CHUNK,), jnp.int32) for _ in range(NBUF)]
        + [pltpu.SemaphoreType.DMA for _ in range(3 * NBUF + 1)],
    )
    def k(x_hbm, e_hbm, zero_hbm, out_hbm, src_all, acc, *bufs):
        rows = bufs[:NBUF]
        dst_v = bufs[NBUF:2 * NBUF]
        gsem = bufs[2 * NBUF:3 * NBUF]
        dsem = bufs[3 * NBUF:4 * NBUF]
        ssem = bufs[4 * NBUF:5 * NBUF]
        zsem = bufs[5 * NBUF]
        c = lax.axis_index("c")
        s = lax.axis_index("s")
        w = s * NC + c

        zcopy = pltpu.async_copy(zero_hbm, acc.at[pl.ds(s * RPT, RPT)], zsem)
        ebase = w * EPW
        pltpu.sync_copy(e_hbm.at[pl.ds(ebase, EPW)], src_all)

        def start_chunk(j, u, reuse=True):
            if reuse:
                pltpu.make_async_copy(
                    rows[u], acc.at[dst_v[u]], ssem[u]).wait()
            base = pl.multiple_of(j * CHUNK, 8)
            pltpu.async_copy(e_hbm.at[pl.ds(E + ebase + base, CHUNK)],
                             dst_v[u], dsem[u])
            pltpu.async_copy(x_hbm.at[src_all.at[pl.ds(base, CHUNK)]],
                             rows[u], gsem[u])

        def finish_chunk(u):
            pltpu.make_async_copy(
                e_hbm.at[pl.ds(0, CHUNK)], dst_v[u], dsem[u]).wait()
            pltpu.make_async_copy(
                x_hbm.at[pl.ds(0, CHUNK)], rows[u], gsem[u]).wait()
            pltpu.async_copy(rows[u], acc.at[dst_v[u]], ssem[u], add=True)

        for u in range(NBUF):
            start_chunk(u, u, reuse=False)
        zcopy.wait()
        plsc.subcore_barrier()

        nfull = (NCHUNK - NBUF) // NBUF

        def body(i, carry):
            for u in range(NBUF):
                j = i * NBUF + u
                finish_chunk(u)
                start_chunk(j + NBUF, u)
            return carry

        lax.fori_loop(0, nfull, body, 0)
        for j in range(nfull * NBUF, NCHUNK):
            u = j % NBUF
            finish_chunk(u)
            if j + NBUF < NCHUNK:
                start_chunk(j + NBUF, u)
        for u in range(NBUF):
            pltpu.make_async_copy(rows[u], acc.at[dst_v[u]], ssem[u]).wait()
        plsc.subcore_barrier()

        @pl.when(s < NS - 1)
        def _():
            pltpu.sync_copy(acc.at[pl.ds(s * RPT, RPT)],
                            out_hbm.at[pl.ds(c * N + s * RPT, RPT)])

        @pl.when(s == NS - 1)
        def _():
            pltpu.sync_copy(acc.at[pl.ds((NS - 1) * RPT, LAST)],
                            out_hbm.at[pl.ds(c * N + (NS - 1) * RPT, LAST)])

    return k(x, edges, zero_init)


BM = 5000


def _tc_tail(x, partials, W, b2):
    def body(x_ref, p0_ref, p1_ref, w_ref, b_ref, o_ref):
        a = p0_ref[...] + p1_ref[...]
        h = jnp.dot(a, w_ref[...], preferred_element_type=jnp.float32)
        o_ref[...] = x_ref[...] + jnp.maximum(h + b_ref[...], 0.0)

    return pl.pallas_call(
        body,
        grid=(N // BM,),
        in_specs=[
            pl.BlockSpec((BM, D), lambda i: (i, 0)),
            pl.BlockSpec((BM, D), lambda i: (i, 0)),
            pl.BlockSpec((BM, D), lambda i: (i + N // BM, 0)),
            pl.BlockSpec((D, D), lambda i: (0, 0)),
            pl.BlockSpec((1, D), lambda i: (0, 0)),
        ],
        out_specs=pl.BlockSpec((BM, D), lambda i: (i, 0)),
        out_shape=jax.ShapeDtypeStruct((N, D), jnp.float32),
    )(x, partials, partials, W, b2)


def kernel(input, edge_index, cell_dropout, layer_dropout, node_lastlayer,
           stage1_flag, W, b):
    edges = edge_index.reshape(2 * E)
    zero_init = jnp.zeros((RPT, D), dtype=jnp.float32)

    partials = _sc_segment_sum(input, edges, zero_init)
    return _tc_tail(input, partials, W, b.reshape(1, D))
